# baseline (speedup 1.0000x reference)
import functools

import jax
import jax.numpy as jnp
from jax import lax
from jax.experimental import pallas as pl
from jax.experimental.pallas import tpu as pltpu
from jax.experimental.pallas import tpu_sc as plsc

N, K, E, O, I = 2048, 2, 64, 128, 128
M = O * I
NC, NS = 2, 16
NW = NC * NS
CW = M // NW
CH = 32
NCH = N // CH
LANES = 16


def _splat(val, dtype):
    return lax.broadcast_in_dim(jnp.asarray(val, dtype), (LANES,), ())


def _sc_weight_body(wp_hbm, wi_hbm, bank_hbm, out_hbm,
                    bank_v, idx_v, prob_v, outb0, outb1,
                    sem_b0, sem_b1, sem_in):
    wid = lax.axis_index("s") * NC + lax.axis_index("c")
    col0 = wid * CW

    pltpu.async_copy(bank_hbm.at[:, pl.ds(col0, CW)], bank_v, sem_in).wait()
    pltpu.async_copy(wi_hbm, idx_v, sem_in).wait()
    pltpu.async_copy(wp_hbm, prob_v, sem_in).wait()

    def chunk(ch, outb, sem, first):
        t0 = ch * CH

        def token(t, carry):
            ti = (t0 + t) * K
            i0v = plsc.load_gather(idx_v, [_splat(ti, jnp.int32)])
            i1v = plsc.load_gather(idx_v, [_splat(ti + 1, jnp.int32)])
            p0v = plsc.load_gather(prob_v, [_splat(ti, jnp.int32)])
            p1v = plsc.load_gather(prob_v, [_splat(ti + 1, jnp.int32)])
            i0 = jnp.max(i0v)
            i1 = jnp.max(i1v)
            for c in range(CW // LANES):
                va = bank_v[i0, pl.ds(c * LANES, LANES)]
                vb = bank_v[i1, pl.ds(c * LANES, LANES)]
                outb[t, pl.ds(c * LANES, LANES)] = p0v * va + p1v * vb
            return carry

        @pl.when(jnp.logical_not(first))
        def _():
            pltpu.make_async_copy(
                outb, out_hbm.at[pl.ds(t0, CH), pl.ds(col0, CW)], sem).wait()

        lax.fori_loop(0, CH, token, 0, unroll=1)
        pltpu.async_copy(
            outb, out_hbm.at[pl.ds(t0, CH), pl.ds(col0, CW)], sem)

    def pair(g, carry):
        chunk(g * 2, outb0, sem_b0, g == 0)
        chunk(g * 2 + 1, outb1, sem_b1, g == 0)
        return carry

    lax.fori_loop(0, NCH // 2, pair, 0, unroll=1)
    pltpu.make_async_copy(
        outb0, out_hbm.at[pl.ds(0, CH), pl.ds(col0, CW)], sem_b0).wait()
    pltpu.make_async_copy(
        outb1, out_hbm.at[pl.ds(0, CH), pl.ds(col0, CW)], sem_b1).wait()


_sc_weight = functools.partial(
    pl.kernel,
    out_type=jax.ShapeDtypeStruct((N, M), jnp.float32),
    mesh=plsc.VectorSubcoreMesh(core_axis_name="c", subcore_axis_name="s"),
    compiler_params=pltpu.CompilerParams(needs_layout_passes=False),
    scratch_types=[
        pltpu.VMEM((E, CW), jnp.float32),
        pltpu.VMEM((N * K,), jnp.int32),
        pltpu.VMEM((N * K,), jnp.float32),
        pltpu.VMEM((CH, CW), jnp.float32),
        pltpu.VMEM((CH, CW), jnp.float32),
        pltpu.SemaphoreType.DMA,
        pltpu.SemaphoreType.DMA,
        pltpu.SemaphoreType.DMA,
    ],
)(_sc_weight_body)


def _tc_bias_kernel(bp_ref, bi_ref, bbank_ref, bout_ref):
    bp = bp_ref[...]
    bi = bi_ref[...]
    iota = lax.broadcasted_iota(jnp.int32, (N, E), 1)
    sb = (bp[:, 0:1] * (bi[:, 0:1] == iota).astype(jnp.float32)
          + bp[:, 1:2] * (bi[:, 1:2] == iota).astype(jnp.float32))
    bout_ref[...] = jnp.dot(sb, bbank_ref[...],
                            preferred_element_type=jnp.float32)


def kernel(weight_probs, weight_indices, bias_probs, bias_indices,
           weight_bank, bias_bank):
    wi = weight_indices.astype(jnp.int32).reshape(N * K)
    bi = bias_indices.astype(jnp.int32)
    wp = weight_probs.reshape(N * K)
    bank2d = weight_bank.reshape(E, M)

    out2d = _sc_weight(wp, wi, bank2d)

    bout = pl.pallas_call(
        _tc_bias_kernel,
        out_shape=jax.ShapeDtypeStruct((N, O), jnp.float32),
    )(bias_probs, bi, bias_bank)

    return out2d.reshape(N, O, I), bout


# speedup vs baseline: 1.9036x; 1.9036x over previous
import functools

import jax
import jax.numpy as jnp
from jax import lax
from jax.experimental import pallas as pl
from jax.experimental.pallas import tpu as pltpu
from jax.experimental.pallas import tpu_sc as plsc

N, K, E, O, I = 2048, 2, 64, 128, 128
M = O * I
NC, NS = 2, 16
NW = NC * NS
CW = M // NW
CH = 32
NCH = N // CH
LANES = 16


def _splat(val, dtype):
    return lax.broadcast_in_dim(jnp.asarray(val, dtype), (LANES,), ())


def _sc_weight_body(wp_hbm, wi_hbm, bank_hbm, out_hbm,
                    bank_v, idx_v, prob_v, outb0, outb1,
                    sem_b0, sem_b1, sem_in):
    wid = lax.axis_index("s") * NC + lax.axis_index("c")
    col0 = wid * CW

    pltpu.async_copy(bank_hbm.at[:, pl.ds(col0, CW)], bank_v, sem_in).wait()
    pltpu.async_copy(wi_hbm, idx_v, sem_in).wait()
    pltpu.async_copy(wp_hbm, prob_v, sem_in).wait()

    def chunk(ch, outb, sem, first):
        t0 = ch * CH

        def token(t):
            ti = (t0 + t) * K
            i0v = plsc.load_gather(idx_v, [_splat(ti, jnp.int32)])
            i1v = plsc.load_gather(idx_v, [_splat(ti + 1, jnp.int32)])
            p0v = plsc.load_gather(prob_v, [_splat(ti, jnp.int32)])
            p1v = plsc.load_gather(prob_v, [_splat(ti + 1, jnp.int32)])
            i0 = jnp.max(i0v)
            i1 = jnp.max(i1v)
            for c in range(CW // LANES):
                va = bank_v[i0, pl.ds(c * LANES, LANES)]
                vb = bank_v[i1, pl.ds(c * LANES, LANES)]
                outb[t, pl.ds(c * LANES, LANES)] = p0v * va + p1v * vb

        @pl.when(jnp.logical_not(first))
        def _():
            pltpu.make_async_copy(
                outb, out_hbm.at[pl.ds(t0, CH), pl.ds(col0, CW)], sem).wait()

        plsc.parallel_loop(0, CH, 1, unroll=4)(token)
        pltpu.async_copy(
            outb, out_hbm.at[pl.ds(t0, CH), pl.ds(col0, CW)], sem)

    def pair(g, carry):
        chunk(g * 2, outb0, sem_b0, g == 0)
        chunk(g * 2 + 1, outb1, sem_b1, g == 0)
        return carry

    lax.fori_loop(0, NCH // 2, pair, 0, unroll=1)
    pltpu.make_async_copy(
        outb0, out_hbm.at[pl.ds(0, CH), pl.ds(col0, CW)], sem_b0).wait()
    pltpu.make_async_copy(
        outb1, out_hbm.at[pl.ds(0, CH), pl.ds(col0, CW)], sem_b1).wait()


_sc_weight = functools.partial(
    pl.kernel,
    out_type=jax.ShapeDtypeStruct((N, M), jnp.float32),
    mesh=plsc.VectorSubcoreMesh(core_axis_name="c", subcore_axis_name="s"),
    compiler_params=pltpu.CompilerParams(needs_layout_passes=False),
    scratch_types=[
        pltpu.VMEM((E, CW), jnp.float32),
        pltpu.VMEM((N * K,), jnp.int32),
        pltpu.VMEM((N * K,), jnp.float32),
        pltpu.VMEM((CH, CW), jnp.float32),
        pltpu.VMEM((CH, CW), jnp.float32),
        pltpu.SemaphoreType.DMA,
        pltpu.SemaphoreType.DMA,
        pltpu.SemaphoreType.DMA,
    ],
)(_sc_weight_body)


def _tc_bias_kernel(bp_ref, bi_ref, bbank_ref, bout_ref):
    bp = bp_ref[...]
    bi = bi_ref[...]
    iota = lax.broadcasted_iota(jnp.int32, (N, E), 1)
    sb = (bp[:, 0:1] * (bi[:, 0:1] == iota).astype(jnp.float32)
          + bp[:, 1:2] * (bi[:, 1:2] == iota).astype(jnp.float32))
    bout_ref[...] = jnp.dot(sb, bbank_ref[...],
                            preferred_element_type=jnp.float32)


def kernel(weight_probs, weight_indices, bias_probs, bias_indices,
           weight_bank, bias_bank):
    wi = weight_indices.astype(jnp.int32).reshape(N * K)
    bi = bias_indices.astype(jnp.int32)
    wp = weight_probs.reshape(N * K)
    bank2d = weight_bank.reshape(E, M)

    out2d = _sc_weight(wp, wi, bank2d)

    bout = pl.pallas_call(
        _tc_bias_kernel,
        out_shape=jax.ShapeDtypeStruct((N, O), jnp.float32),
    )(bias_probs, bi, bias_bank)

    return out2d.reshape(N, O, I), bout
